# baseline (speedup 1.0000x reference)
import functools

import jax
import jax.numpy as jnp
from jax import lax
from jax.experimental import layout as jlayout
from jax.experimental import pallas as pl
from jax.experimental.pallas import tpu as pltpu
from jax.experimental.pallas import tpu_sc as plsc

VOCAB = 1000000
D_MODEL = 64
PADDING_IDX = 0
BATCH = 4096
SEQ_LEN = 200

NUM_CORES = 2
NUM_SUBCORES = 16
NW = NUM_CORES * NUM_SUBCORES

TOTAL = BATCH * SEQ_LEN
IDX_W = 128
ROWS_TOTAL = TOTAL // IDX_W
ROWS_PER_W = ROWS_TOTAL // NW
G = 5
CHUNK = G * IDX_W
N_CHUNKS = ROWS_PER_W // G


def _sc_gather(tok2d, table):
    mesh = plsc.VectorSubcoreMesh(
        core_axis_name="c", subcore_axis_name="s",
        num_cores=NUM_CORES, num_subcores=NUM_SUBCORES)

    @functools.partial(
        pl.kernel,
        out_type=jax.ShapeDtypeStruct((TOTAL, D_MODEL), jnp.float32),
        mesh=mesh,
        scratch_types=[
            pltpu.VMEM((ROWS_PER_W, IDX_W), jnp.int32),
            pltpu.VMEM((CHUNK, D_MODEL), jnp.float32),
            pltpu.VMEM((CHUNK, D_MODEL), jnp.float32),
            pltpu.SemaphoreType.DMA,
            pltpu.SemaphoreType.DMA,
            pltpu.SemaphoreType.DMA,
            pltpu.SemaphoreType.DMA,
        ],
        compiler_params=pltpu.CompilerParams(
            use_tc_tiling_on_sc=False, needs_layout_passes=True),
    )
    def k(tok_hbm, table_hbm, out_hbm, idx_v, rows0, rows1, g0, g1, o0, o1):
        wid = lax.axis_index("s") * NUM_CORES + lax.axis_index("c")
        tok_base = wid * ROWS_PER_W * IDX_W
        rows = (rows0, rows1)
        gsem = (g0, g1)
        osem = (o0, o1)

        pltpu.sync_copy(tok_hbm.at[pl.ds(wid * ROWS_PER_W, ROWS_PER_W)], idx_v)

        def issue_gathers(c, b):
            for j in range(G):
                pltpu.async_copy(
                    table_hbm.at[idx_v.at[c * G + j]],
                    rows[b].at[pl.ds(j * IDX_W, IDX_W)],
                    gsem[b])

        def wait_gathers(b):
            pltpu.make_async_copy(
                table_hbm.at[pl.ds(0, CHUNK)], rows[b], gsem[b]).wait()

        def issue_writeback(c, b):
            pltpu.async_copy(
                rows[b], out_hbm.at[pl.ds(tok_base + c * CHUNK, CHUNK)],
                osem[b])

        def wait_writeback(b):
            pltpu.make_async_copy(
                rows[b], out_hbm.at[pl.ds(0, CHUNK)], osem[b]).wait()

        issue_gathers(0, 0)

        @pl.loop(0, N_CHUNKS, step=2)
        def _group(c0):
            c1 = c0 + 1
            @pl.when(c0 > 0)
            def _():
                wait_writeback(1)
            issue_gathers(c1, 1)
            wait_gathers(0)
            issue_writeback(c0, 0)

            @pl.when(c0 + 2 < N_CHUNKS)
            def _():
                wait_writeback(0)
                issue_gathers(c0 + 2, 0)
            wait_gathers(1)
            issue_writeback(c1, 1)

        wait_writeback(0)
        wait_writeback(1)

    return k(tok2d, table)


def _masks_kernel(tok_ref, pad_ref, seq_ref):
    pad_ref[...] = tok_ref[...] == PADDING_IDX
    r = lax.broadcasted_iota(jnp.int32, (SEQ_LEN, SEQ_LEN), 0)
    c = lax.broadcasted_iota(jnp.int32, (SEQ_LEN, SEQ_LEN), 1)
    seq_ref[...] = c > r


def _masks(tokens):
    return pl.pallas_call(
        _masks_kernel,
        out_shape=(
            jax.ShapeDtypeStruct((BATCH, SEQ_LEN), jnp.bool_),
            jax.ShapeDtypeStruct((SEQ_LEN, SEQ_LEN), jnp.bool_),
        ),
    )(tokens)


def kernel(tokens, table):
    tokens = tokens.astype(jnp.int32)
    tok2d = tokens.reshape(ROWS_TOTAL, IDX_W)
    feats = _sc_gather(tok2d, table).reshape(BATCH, SEQ_LEN, D_MODEL)
    feats = jax.device_put(
        feats,
        jlayout.Format(
            jlayout.Layout(major_to_minor=(0, 1, 2)),
            jax.sharding.SingleDeviceSharding(jax.devices()[0])))
    pad, seq = _masks(tokens)
    return (feats, (pad[:, None, None, :], seq))


# speedup vs baseline: 1.3307x; 1.3307x over previous
import functools

import jax
import jax.numpy as jnp
from jax import lax
from jax.experimental import pallas as pl
from jax.experimental.pallas import tpu as pltpu
from jax.experimental.pallas import tpu_sc as plsc

VOCAB = 1000000
D_MODEL = 64
PADDING_IDX = 0
BATCH = 4096
SEQ_LEN = 200

NUM_CORES = 2
NUM_SUBCORES = 16
NW = NUM_CORES * NUM_SUBCORES

TOTAL = BATCH * SEQ_LEN
IDX_W = 128
ROWS_TOTAL = TOTAL // IDX_W
ROWS_PER_W = ROWS_TOTAL // NW
G = 5
CHUNK = G * IDX_W
N_CHUNKS = ROWS_PER_W // G


def _sc_gather(tok2d, table):
    mesh = plsc.VectorSubcoreMesh(
        core_axis_name="c", subcore_axis_name="s",
        num_cores=NUM_CORES, num_subcores=NUM_SUBCORES)

    @functools.partial(
        pl.kernel,
        out_type=jax.ShapeDtypeStruct((TOTAL, 2 * D_MODEL), jnp.float32),
        mesh=mesh,
        scratch_types=[
            pltpu.VMEM((ROWS_PER_W, IDX_W), jnp.int32),
            pltpu.VMEM((CHUNK, D_MODEL), jnp.float32),
            pltpu.VMEM((CHUNK, D_MODEL), jnp.float32),
            pltpu.SemaphoreType.DMA,
            pltpu.SemaphoreType.DMA,
            pltpu.SemaphoreType.DMA,
            pltpu.SemaphoreType.DMA,
        ],
        compiler_params=pltpu.CompilerParams(
            use_tc_tiling_on_sc=False, needs_layout_passes=True),
    )
    def k(tok_hbm, table_hbm, out_hbm, idx_v, rows0, rows1, g0, g1, o0, o1):
        wid = lax.axis_index("s") * NUM_CORES + lax.axis_index("c")
        tok_base = wid * ROWS_PER_W * IDX_W
        rows = (rows0, rows1)
        gsem = (g0, g1)
        osem = (o0, o1)

        pltpu.sync_copy(tok_hbm.at[pl.ds(wid * ROWS_PER_W, ROWS_PER_W)], idx_v)

        def issue_gathers(c, b):
            for j in range(G):
                pltpu.async_copy(
                    table_hbm.at[idx_v.at[c * G + j]],
                    rows[b].at[pl.ds(j * IDX_W, IDX_W)],
                    gsem[b])

        def wait_gathers(b):
            pltpu.make_async_copy(
                table_hbm.at[pl.ds(0, CHUNK)], rows[b], gsem[b]).wait()

        def issue_writeback(c, b):
            pltpu.async_copy(
                rows[b],
                out_hbm.at[pl.ds(tok_base + c * CHUNK, CHUNK),
                           pl.ds(0, D_MODEL)],
                osem[b])

        def wait_writeback(b):
            pltpu.make_async_copy(
                rows[b], out_hbm.at[pl.ds(0, CHUNK), pl.ds(0, D_MODEL)],
                osem[b]).wait()

        issue_gathers(0, 0)

        @pl.loop(0, N_CHUNKS, step=2)
        def _group(c0):
            c1 = c0 + 1
            @pl.when(c0 > 0)
            def _():
                wait_writeback(1)
            issue_gathers(c1, 1)
            wait_gathers(0)
            issue_writeback(c0, 0)

            @pl.when(c0 + 2 < N_CHUNKS)
            def _():
                wait_writeback(0)
                issue_gathers(c0 + 2, 0)
            wait_gathers(1)
            issue_writeback(c1, 1)

        wait_writeback(0)
        wait_writeback(1)

    return k(tok2d, table)


def _masks_kernel(tok_ref, pad_ref, seq_ref):
    pad_ref[...] = tok_ref[...] == PADDING_IDX
    r = lax.broadcasted_iota(jnp.int32, (SEQ_LEN, SEQ_LEN), 0)
    c = lax.broadcasted_iota(jnp.int32, (SEQ_LEN, SEQ_LEN), 1)
    seq_ref[...] = c > r


def _masks(tokens):
    return pl.pallas_call(
        _masks_kernel,
        out_shape=(
            jax.ShapeDtypeStruct((BATCH, SEQ_LEN), jnp.bool_),
            jax.ShapeDtypeStruct((SEQ_LEN, SEQ_LEN), jnp.bool_),
        ),
    )(tokens)


def kernel(tokens, table):
    tokens = tokens.astype(jnp.int32)
    tok2d = tokens.reshape(ROWS_TOTAL, IDX_W)
    padded = _sc_gather(tok2d, table)
    feats = padded[:, :D_MODEL].reshape(BATCH, SEQ_LEN, D_MODEL)
    pad, seq = _masks(tokens)
    return (feats, (pad[:, None, None, :], seq))
